# baseline (speedup 1.0000x reference)
import math

import jax
import jax.numpy as jnp
from jax import lax
from jax.experimental import pallas as pl
from jax.experimental.pallas import tpu as pltpu
from jax.experimental.pallas import tpu_sc as plsc

LANES = 16
NCORES = 2
NSUB = 16
NW = NCORES * NSUB
CH = 16


def _vgather(x, idx):
    return lax.gather(
        x, idx[:, None],
        lax.GatherDimensionNumbers(
            offset_dims=(), collapsed_slice_dims=(0,), start_index_map=(0,)),
        (1,),
        mode=lax.GatherScatterMode.PROMISE_IN_BOUNDS)


def _project_tc(xh, W, b, inv_sqrt):
    N, D = xh.shape
    C4 = W.shape[1]
    C = C4 // 4
    BN = 1000

    def body(x_ref, w_ref, b_ref, q_ref, kv_ref, s_ref):
        acc = jnp.dot(x_ref[...], w_ref[...],
                      preferred_element_type=jnp.float32) + b_ref[...]
        q_ref[...] = acc[:, 0:C] * inv_sqrt
        kv_ref[...] = acc[:, C:3 * C]
        s_ref[...] = acc[:, 3 * C:4 * C]

    return pl.pallas_call(
        body,
        grid=(N // BN,),
        in_specs=[
            pl.BlockSpec((BN, D), lambda i: (i, 0)),
            pl.BlockSpec((D, C4), lambda i: (0, 0)),
            pl.BlockSpec((1, C4), lambda i: (0, 0)),
        ],
        out_specs=[
            pl.BlockSpec((BN, C), lambda i: (i, 0)),
            pl.BlockSpec((BN, 2 * C), lambda i: (i, 0)),
            pl.BlockSpec((BN, C), lambda i: (i, 0)),
        ],
        out_shape=[
            jax.ShapeDtypeStruct((N, C), jnp.float32),
            jax.ShapeDtypeStruct((N, 2 * C), jnp.float32),
            jax.ShapeDtypeStruct((N, C), jnp.float32),
        ],
    )(xh, W, b)


def _sum32_tc(s_all):
    def body(s_ref, o_ref):
        o_ref[...] = jnp.sum(s_ref[...], axis=0, keepdims=True) + 1e-16

    return pl.pallas_call(
        body,
        out_shape=jax.ShapeDtypeStruct((1, s_all.shape[1]), jnp.float32),
    )(s_all)


def _norm_project_tc(o_part, s_col, skip, W, b, inv_sqrt):
    _, N, C = o_part.shape
    C4 = W.shape[1]
    BN = 1000

    def body(o_ref, sc_ref, sk_ref, w_ref, b_ref, q_ref, kv_ref, s_ref):
        att = (o_ref[0] + o_ref[1]) / sc_ref[...]
        h = jax.nn.relu(att + sk_ref[...])
        acc = jnp.dot(h, w_ref[...],
                      preferred_element_type=jnp.float32) + b_ref[...]
        q_ref[...] = acc[:, 0:C] * inv_sqrt
        kv_ref[...] = acc[:, C:3 * C]
        s_ref[...] = acc[:, 3 * C:4 * C]

    return pl.pallas_call(
        body,
        grid=(N // BN,),
        in_specs=[
            pl.BlockSpec((2, BN, C), lambda i: (0, i, 0)),
            pl.BlockSpec((BN, 1), lambda i: (i, 0)),
            pl.BlockSpec((BN, C), lambda i: (i, 0)),
            pl.BlockSpec((C, C4), lambda i: (0, 0)),
            pl.BlockSpec((1, C4), lambda i: (0, 0)),
        ],
        out_specs=[
            pl.BlockSpec((BN, C), lambda i: (i, 0)),
            pl.BlockSpec((BN, 2 * C), lambda i: (i, 0)),
            pl.BlockSpec((BN, C), lambda i: (i, 0)),
        ],
        out_shape=[
            jax.ShapeDtypeStruct((N, C), jnp.float32),
            jax.ShapeDtypeStruct((N, 2 * C), jnp.float32),
            jax.ShapeDtypeStruct((N, C), jnp.float32),
        ],
    )(o_part, s_col, skip, W, b)


def _combine_tc(o_part, s_col, skip):
    _, N, C = o_part.shape
    BN = 1000

    def body(o_ref, sc_ref, sk_ref, out_ref):
        att = (o_ref[0] + o_ref[1]) / sc_ref[...]
        out_ref[...] = jax.nn.relu(att + sk_ref[...])

    return pl.pallas_call(
        body,
        grid=(N // BN,),
        in_specs=[
            pl.BlockSpec((2, BN, C), lambda i: (0, i, 0)),
            pl.BlockSpec((BN, 1), lambda i: (i, 0)),
            pl.BlockSpec((BN, C), lambda i: (i, 0)),
        ],
        out_specs=pl.BlockSpec((BN, C), lambda i: (i, 0)),
        out_shape=jax.ShapeDtypeStruct((N, C), jnp.float32),
    )(o_part, s_col, skip)


def _edge_layer_sc(qm, kvm, dstv, srcv, zeros_h):
    N, C = qm.shape
    E = dstv.shape[0]
    EPW = E // NW
    NCH = EPW // CH
    NG = CH // LANES
    NR = C // LANES
    SPAN = ((N + NSUB - 1) // NSUB + 7) // 8 * 8
    assert NCH % 2 == 1

    mesh = plsc.VectorSubcoreMesh(core_axis_name="c", subcore_axis_name="s")

    def body(q_hbm, kv_hbm, dst_hbm, src_hbm, z_hbm, out_part, s_all,
             dst_w, src_w, s_local, qb, kvb, vob, idxd,
             gsem0, gsem1, ssem0, ssem1, out_shared):
        cid = lax.axis_index("c")
        sid = lax.axis_index("s")
        wid = sid * NCORES + cid
        base = wid * EPW
        gsems = (gsem0, gsem1)
        ssems = (ssem0, ssem1)

        iota = lax.iota(jnp.int32, LANES)
        ziota = iota * 0
        zero = jnp.zeros((LANES,), jnp.float32)

        def fire_gathers(b, cc):
            cb = cc * CH
            pltpu.async_copy(
                q_hbm.at[dst_w.at[pl.ds(cb, CH)]], qb.at[b], gsems[b])
            pltpu.async_copy(
                kv_hbm.at[src_w.at[pl.ds(cb, CH)]], kvb.at[b], gsems[b])

        def wait_gathers(b):
            pltpu.make_async_copy(
                q_hbm.at[dst_w.at[pl.ds(0, CH)]], qb.at[b], gsems[b]).wait()
            pltpu.make_async_copy(
                kv_hbm.at[src_w.at[pl.ds(0, CH)]], kvb.at[b], gsems[b]).wait()

        def fire_scatter(b):
            pltpu.async_copy(
                vob.at[b], out_shared.at[idxd.at[b]], ssems[b], add=True)

        def wait_scatter(b):
            pltpu.make_async_copy(
                vob.at[b], out_shared.at[idxd.at[b]], ssems[b]).wait()

        def compute(b, cc):
            cb = cc * CH
            qsl = qb.at[b]
            kvsl = kvb.at[b]
            vosl = vob.at[b]
            def dot_blk(jb, accs):
                a0, a1, a2, a3 = accs
                j0 = jb * LANES
                for u in range(0, LANES, 4):
                    cj = ziota + (j0 + u)
                    a0 = a0 + (plsc.load_gather(qsl, [iota, cj])
                               * plsc.load_gather(kvsl, [iota, cj]))
                    cj = ziota + (j0 + u + 1)
                    a1 = a1 + (plsc.load_gather(qsl, [iota, cj])
                               * plsc.load_gather(kvsl, [iota, cj]))
                    cj = ziota + (j0 + u + 2)
                    a2 = a2 + (plsc.load_gather(qsl, [iota, cj])
                               * plsc.load_gather(kvsl, [iota, cj]))
                    cj = ziota + (j0 + u + 3)
                    a3 = a3 + (plsc.load_gather(qsl, [iota, cj])
                               * plsc.load_gather(kvsl, [iota, cj]))
                return a0, a1, a2, a3

            a0, a1, a2, a3 = lax.fori_loop(
                0, C // LANES, dot_blk, (zero, zero, zero, zero))
            ev = jnp.exp((a0 + a1) + (a2 + a3))
            ks = dst_w[pl.ds(cb, LANES)]
            vs = ev
            mn = iota
            for r in range(1, LANES):
                perm = (iota + r) & (LANES - 1)
                same = _vgather(ks, perm) == ks
                vs = vs + jnp.where(same, _vgather(ev, perm), 0.0)
                mn = jnp.minimum(mn, jnp.where(same, perm, LANES))
            plsc.addupdate_scatter(s_local, [ks], vs, mask=mn == iota)
            idxd[b, pl.ds(0, LANES)] = ks
            def scale_blk(jb, scarry):
                j0 = jb * LANES
                for u in range(LANES):
                    cj = ziota + (j0 + u)
                    plsc.store_scatter(
                        vosl, [iota, cj],
                        plsc.load_gather(kvsl, [iota, cj + C]) * ev)
                return scarry

            lax.fori_loop(0, C // LANES, scale_blk, 0)

        pltpu.sync_copy(dst_hbm.at[pl.ds(base, EPW)], dst_w)
        pltpu.sync_copy(src_hbm.at[pl.ds(base, EPW)], src_w)

        def zloop(i, carry):
            s_local[pl.ds(i * LANES, LANES)] = zero
            return carry

        lax.fori_loop(0, N // LANES, zloop, 0)
        stripe = jnp.minimum(sid * SPAN, N - SPAN)
        pltpu.sync_copy(z_hbm.at[pl.ds(stripe, SPAN)],
                        out_shared.at[pl.ds(stripe, SPAN)])
        fire_gathers(0, 0)
        plsc.subcore_barrier()

        def pair(i, carry):
            for b in (0, 1):
                cc = 2 * i + b
                fire_gathers(b ^ 1, cc + 1)
                wait_gathers(b)

                @pl.when(cc >= 2)
                def _():
                    wait_scatter(b)

                compute(b, cc)
                fire_scatter(b)
            return carry

        lax.fori_loop(0, (NCH - 1) // 2, pair, 0)

        wait_gathers(0)
        wait_scatter(0)
        compute(0, NCH - 1)
        fire_scatter(0)
        wait_scatter(1)
        wait_scatter(0)
        plsc.subcore_barrier()
        pltpu.sync_copy(s_local, s_all.at[wid])
        pltpu.sync_copy(out_shared.at[pl.ds(stripe, SPAN)],
                        out_part.at[cid, pl.ds(stripe, SPAN)])

    return pl.kernel(
        body,
        out_type=[
            jax.ShapeDtypeStruct((NCORES, N, C), jnp.float32),
            jax.ShapeDtypeStruct((NW, N), jnp.float32),
        ],
        mesh=mesh,
        scratch_types=[
            pltpu.VMEM((EPW,), jnp.int32),
            pltpu.VMEM((EPW,), jnp.int32),
            pltpu.VMEM((N,), jnp.float32),
            pltpu.VMEM((2, CH, C), jnp.float32),
            pltpu.VMEM((2, CH, 2 * C), jnp.float32),
            pltpu.VMEM((2, CH, C), jnp.float32),
            pltpu.VMEM((2, CH), jnp.int32),
            pltpu.SemaphoreType.DMA,
            pltpu.SemaphoreType.DMA,
            pltpu.SemaphoreType.DMA,
            pltpu.SemaphoreType.DMA,
            pltpu.VMEM_SHARED((N, C), jnp.float32),
        ],
        compiler_params=pltpu.CompilerParams(needs_layout_passes=False),
    )(qm, kvm, dstv, srcv, zeros_h)


def kernel(x, edge_index, Wq0, bq0, Wk0, bk0, Wv0, bv0, Ws0, bs0,
           Wq1, bq1, Wk1, bk1, Wv1, bv1, Ws1, bs1):
    src = edge_index[0]
    dst = edge_index[1]
    N, _ = x.shape
    C = Wq0.shape[1]
    inv_sqrt = 1.0 / math.sqrt(C)

    W0 = jnp.concatenate([Wq0, Wk0, Wv0, Ws0], axis=1)
    b0 = jnp.concatenate([bq0, bk0, bv0, bs0]).reshape(1, -1)
    W1 = jnp.concatenate([Wq1, Wk1, Wv1, Ws1], axis=1)
    b1 = jnp.concatenate([bq1, bk1, bv1, bs1]).reshape(1, -1)
    zeros_h = jnp.zeros((N, C), jnp.float32)

    q0, kv0, sk0 = _project_tc(x, W0, b0, inv_sqrt)
    op0, s_all0 = _edge_layer_sc(q0, kv0, dst, src, zeros_h)
    s_col0 = _sum32_tc(s_all0).reshape(N, 1)
    q1, kv1, sk1 = _norm_project_tc(op0, s_col0, sk0, W1, b1, inv_sqrt)
    op1, s_all1 = _edge_layer_sc(q1, kv1, dst, src, zeros_h)
    s_col1 = _sum32_tc(s_all1).reshape(N, 1)
    return _combine_tc(op1, s_col1, sk1)


# speedup vs baseline: 4.6840x; 4.6840x over previous
import math

import jax
import jax.numpy as jnp
from jax import lax
from jax.experimental import pallas as pl
from jax.experimental.pallas import tpu as pltpu
from jax.experimental.pallas import tpu_sc as plsc

LANES = 16
NCORES = 2
NSUB = 16
NW = NCORES * NSUB
CH = 16


def _vgather(x, idx):
    return lax.gather(
        x, idx[:, None],
        lax.GatherDimensionNumbers(
            offset_dims=(), collapsed_slice_dims=(0,), start_index_map=(0,)),
        (1,),
        mode=lax.GatherScatterMode.PROMISE_IN_BOUNDS)


def _project_tc(xh, W, b, inv_sqrt):
    N, D = xh.shape
    C4 = W.shape[1]
    C = C4 // 4
    BN = 1000

    def body(x_ref, w_ref, b_ref, q_ref, kv_ref, s_ref):
        acc = jnp.dot(x_ref[...], w_ref[...],
                      preferred_element_type=jnp.float32) + b_ref[...]
        q_ref[...] = acc[:, 0:C] * inv_sqrt
        kv_ref[...] = acc[:, C:3 * C]
        s_ref[...] = acc[:, 3 * C:4 * C]

    return pl.pallas_call(
        body,
        grid=(N // BN,),
        in_specs=[
            pl.BlockSpec((BN, D), lambda i: (i, 0)),
            pl.BlockSpec((D, C4), lambda i: (0, 0)),
            pl.BlockSpec((1, C4), lambda i: (0, 0)),
        ],
        out_specs=[
            pl.BlockSpec((BN, C), lambda i: (i, 0)),
            pl.BlockSpec((BN, 2 * C), lambda i: (i, 0)),
            pl.BlockSpec((BN, C), lambda i: (i, 0)),
        ],
        out_shape=[
            jax.ShapeDtypeStruct((N, C), jnp.float32),
            jax.ShapeDtypeStruct((N, 2 * C), jnp.float32),
            jax.ShapeDtypeStruct((N, C), jnp.float32),
        ],
    )(xh, W, b)


def _sum32_tc(s_all):
    def body(s_ref, o_ref):
        o_ref[...] = jnp.sum(s_ref[...], axis=0, keepdims=True) + 1e-16

    return pl.pallas_call(
        body,
        out_shape=jax.ShapeDtypeStruct((1, s_all.shape[1]), jnp.float32),
    )(s_all)


def _norm_project_tc(o_part, s_col, skip, W, b, inv_sqrt):
    _, N, C = o_part.shape
    C4 = W.shape[1]
    BN = 1000

    def body(o_ref, sc_ref, sk_ref, w_ref, b_ref, q_ref, kv_ref, s_ref):
        att = (o_ref[0] + o_ref[1]) / sc_ref[...]
        h = jax.nn.relu(att + sk_ref[...])
        acc = jnp.dot(h, w_ref[...],
                      preferred_element_type=jnp.float32) + b_ref[...]
        q_ref[...] = acc[:, 0:C] * inv_sqrt
        kv_ref[...] = acc[:, C:3 * C]
        s_ref[...] = acc[:, 3 * C:4 * C]

    return pl.pallas_call(
        body,
        grid=(N // BN,),
        in_specs=[
            pl.BlockSpec((2, BN, C), lambda i: (0, i, 0)),
            pl.BlockSpec((BN, 1), lambda i: (i, 0)),
            pl.BlockSpec((BN, C), lambda i: (i, 0)),
            pl.BlockSpec((C, C4), lambda i: (0, 0)),
            pl.BlockSpec((1, C4), lambda i: (0, 0)),
        ],
        out_specs=[
            pl.BlockSpec((BN, C), lambda i: (i, 0)),
            pl.BlockSpec((BN, 2 * C), lambda i: (i, 0)),
            pl.BlockSpec((BN, C), lambda i: (i, 0)),
        ],
        out_shape=[
            jax.ShapeDtypeStruct((N, C), jnp.float32),
            jax.ShapeDtypeStruct((N, 2 * C), jnp.float32),
            jax.ShapeDtypeStruct((N, C), jnp.float32),
        ],
    )(o_part, s_col, skip, W, b)


def _combine_tc(o_part, s_col, skip):
    _, N, C = o_part.shape
    BN = 1000

    def body(o_ref, sc_ref, sk_ref, out_ref):
        att = (o_ref[0] + o_ref[1]) / sc_ref[...]
        out_ref[...] = jax.nn.relu(att + sk_ref[...])

    return pl.pallas_call(
        body,
        grid=(N // BN,),
        in_specs=[
            pl.BlockSpec((2, BN, C), lambda i: (0, i, 0)),
            pl.BlockSpec((BN, 1), lambda i: (i, 0)),
            pl.BlockSpec((BN, C), lambda i: (i, 0)),
        ],
        out_specs=pl.BlockSpec((BN, C), lambda i: (i, 0)),
        out_shape=jax.ShapeDtypeStruct((N, C), jnp.float32),
    )(o_part, s_col, skip)


def _edge_layer_sc(qm, kvm, dstv, srcv, zeros_h):
    N, C = qm.shape
    E = dstv.shape[0]
    EPW = E // NW
    NCH = EPW // CH
    NG = CH // LANES
    NR = C // LANES
    SPAN = ((N + NSUB - 1) // NSUB + 7) // 8 * 8
    assert NCH % 2 == 1

    mesh = plsc.VectorSubcoreMesh(core_axis_name="c", subcore_axis_name="s")

    def body(q_hbm, kv_hbm, dst_hbm, src_hbm, z_hbm, out_part, s_all,
             dst_w, src_w, s_local, qb, kvb, vob, idxd,
             gsem0, gsem1, ssem0, ssem1, out_shared):
        cid = lax.axis_index("c")
        sid = lax.axis_index("s")
        wid = sid * NCORES + cid
        base = wid * EPW
        gsems = (gsem0, gsem1)
        ssems = (ssem0, ssem1)

        iota = lax.iota(jnp.int32, LANES)
        ziota = iota * 0
        zero = jnp.zeros((LANES,), jnp.float32)

        def fire_gathers(b, cc):
            cb = cc * CH
            pltpu.async_copy(
                q_hbm.at[dst_w.at[pl.ds(cb, CH)]], qb.at[b], gsems[b])
            pltpu.async_copy(
                kv_hbm.at[src_w.at[pl.ds(cb, CH)]], kvb.at[b], gsems[b])

        def wait_gathers(b):
            pltpu.make_async_copy(
                q_hbm.at[dst_w.at[pl.ds(0, CH)]], qb.at[b], gsems[b]).wait()
            pltpu.make_async_copy(
                kv_hbm.at[src_w.at[pl.ds(0, CH)]], kvb.at[b], gsems[b]).wait()

        def fire_scatter(b):
            pltpu.async_copy(
                vob.at[b], out_shared.at[idxd.at[b]], ssems[b], add=True)

        def wait_scatter(b):
            pltpu.make_async_copy(
                vob.at[b], out_shared.at[idxd.at[b]], ssems[b]).wait()

        def compute(b, cc):
            cb = cc * CH

            def edge4_dot(g, alphas):
                for u in range(4):
                    t = g * 4 + u
                    p = [qb[b, t, pl.ds(r * LANES, LANES)]
                         * kvb[b, t, pl.ds(r * LANES, LANES)]
                         for r in range(NR)]
                    acc = ((p[0] + p[1]) + (p[2] + p[3])) \
                        + ((p[4] + p[5]) + (p[6] + p[7]))
                    for sh in (8, 4, 2, 1):
                        acc = acc + _vgather(acc, iota ^ sh)
                    alphas = jnp.where(iota == t, acc, alphas)
                return alphas

            ev = jnp.exp(lax.fori_loop(0, LANES // 4, edge4_dot, zero))
            ks = dst_w[pl.ds(cb, LANES)]
            vs = ev
            mn = iota
            for r in range(1, LANES):
                perm = (iota + r) & (LANES - 1)
                same = _vgather(ks, perm) == ks
                vs = vs + jnp.where(same, _vgather(ev, perm), 0.0)
                mn = jnp.minimum(mn, jnp.where(same, perm, LANES))
            plsc.addupdate_scatter(s_local, [ks], vs, mask=mn == iota)
            idxd[b, pl.ds(0, LANES)] = ks
            def edge4_scale(g, scarry):
                for u in range(4):
                    t = g * 4 + u
                    av = _vgather(ev, ziota + t)
                    for r in range(NR):
                        vob[b, t, pl.ds(r * LANES, LANES)] = \
                            kvb[b, t, pl.ds(C + r * LANES, LANES)] * av
                return scarry

            lax.fori_loop(0, LANES // 4, edge4_scale, 0)

        pltpu.sync_copy(dst_hbm.at[pl.ds(base, EPW)], dst_w)
        pltpu.sync_copy(src_hbm.at[pl.ds(base, EPW)], src_w)

        def zloop(i, carry):
            s_local[pl.ds(i * LANES, LANES)] = zero
            return carry

        lax.fori_loop(0, N // LANES, zloop, 0)
        stripe = jnp.minimum(sid * SPAN, N - SPAN)
        pltpu.sync_copy(z_hbm.at[pl.ds(stripe, SPAN)],
                        out_shared.at[pl.ds(stripe, SPAN)])
        fire_gathers(0, 0)
        plsc.subcore_barrier()

        def pair(i, carry):
            for b in (0, 1):
                cc = 2 * i + b
                fire_gathers(b ^ 1, cc + 1)
                wait_gathers(b)

                @pl.when(cc >= 2)
                def _():
                    wait_scatter(b)

                compute(b, cc)
                fire_scatter(b)
            return carry

        lax.fori_loop(0, (NCH - 1) // 2, pair, 0)

        wait_gathers(0)
        wait_scatter(0)
        compute(0, NCH - 1)
        fire_scatter(0)
        wait_scatter(1)
        wait_scatter(0)
        plsc.subcore_barrier()
        pltpu.sync_copy(s_local, s_all.at[wid])
        pltpu.sync_copy(out_shared.at[pl.ds(stripe, SPAN)],
                        out_part.at[cid, pl.ds(stripe, SPAN)])

    return pl.kernel(
        body,
        out_type=[
            jax.ShapeDtypeStruct((NCORES, N, C), jnp.float32),
            jax.ShapeDtypeStruct((NW, N), jnp.float32),
        ],
        mesh=mesh,
        scratch_types=[
            pltpu.VMEM((EPW,), jnp.int32),
            pltpu.VMEM((EPW,), jnp.int32),
            pltpu.VMEM((N,), jnp.float32),
            pltpu.VMEM((2, CH, C), jnp.float32),
            pltpu.VMEM((2, CH, 2 * C), jnp.float32),
            pltpu.VMEM((2, CH, C), jnp.float32),
            pltpu.VMEM((2, CH), jnp.int32),
            pltpu.SemaphoreType.DMA,
            pltpu.SemaphoreType.DMA,
            pltpu.SemaphoreType.DMA,
            pltpu.SemaphoreType.DMA,
            pltpu.VMEM_SHARED((N, C), jnp.float32),
        ],
        compiler_params=pltpu.CompilerParams(needs_layout_passes=False),
    )(qm, kvm, dstv, srcv, zeros_h)


def kernel(x, edge_index, Wq0, bq0, Wk0, bk0, Wv0, bv0, Ws0, bs0,
           Wq1, bq1, Wk1, bk1, Wv1, bv1, Ws1, bs1):
    src = edge_index[0]
    dst = edge_index[1]
    N, _ = x.shape
    C = Wq0.shape[1]
    inv_sqrt = 1.0 / math.sqrt(C)

    W0 = jnp.concatenate([Wq0, Wk0, Wv0, Ws0], axis=1)
    b0 = jnp.concatenate([bq0, bk0, bv0, bs0]).reshape(1, -1)
    W1 = jnp.concatenate([Wq1, Wk1, Wv1, Ws1], axis=1)
    b1 = jnp.concatenate([bq1, bk1, bv1, bs1]).reshape(1, -1)
    zeros_h = jnp.zeros((N, C), jnp.float32)

    q0, kv0, sk0 = _project_tc(x, W0, b0, inv_sqrt)
    op0, s_all0 = _edge_layer_sc(q0, kv0, dst, src, zeros_h)
    s_col0 = _sum32_tc(s_all0).reshape(N, 1)
    q1, kv1, sk1 = _norm_project_tc(op0, s_col0, sk0, W1, b1, inv_sqrt)
    op1, s_all1 = _edge_layer_sc(q1, kv1, dst, src, zeros_h)
    s_col1 = _sum32_tc(s_all1).reshape(N, 1)
    return _combine_tc(op1, s_col1, sk1)


# speedup vs baseline: 4.8589x; 1.0373x over previous
import math

import jax
import jax.numpy as jnp
from jax import lax
from jax.experimental import pallas as pl
from jax.experimental.pallas import tpu as pltpu
from jax.experimental.pallas import tpu_sc as plsc

LANES = 16
NCORES = 2
NSUB = 16
NW = NCORES * NSUB
CH = 16


def _unpack(x):
    return plsc.unpack(plsc.bitcast(x, jnp.bfloat16),
                       format=plsc.PackFormat.INTERLEAVED)


def _as_i32(a):
    return lax.bitcast_convert_type(
        a.reshape(*a.shape[:-1], a.shape[-1] // 2, 2), jnp.int32)


def _vgather(x, idx):
    return lax.gather(
        x, idx[:, None],
        lax.GatherDimensionNumbers(
            offset_dims=(), collapsed_slice_dims=(0,), start_index_map=(0,)),
        (1,),
        mode=lax.GatherScatterMode.PROMISE_IN_BOUNDS)


def _project_tc(xh, W, b, inv_sqrt):
    N, D = xh.shape
    C4 = W.shape[1]
    C = C4 // 4
    BN = 1000

    def body(x_ref, w_ref, b_ref, q_ref, kv_ref, s_ref):
        acc = jnp.dot(x_ref[...], w_ref[...],
                      preferred_element_type=jnp.float32) + b_ref[...]
        q_ref[...] = acc[:, 0:C] * inv_sqrt
        kv_ref[...] = acc[:, C:3 * C].astype(jnp.bfloat16)
        s_ref[...] = acc[:, 3 * C:4 * C]

    return pl.pallas_call(
        body,
        grid=(N // BN,),
        in_specs=[
            pl.BlockSpec((BN, D), lambda i: (i, 0)),
            pl.BlockSpec((D, C4), lambda i: (0, 0)),
            pl.BlockSpec((1, C4), lambda i: (0, 0)),
        ],
        out_specs=[
            pl.BlockSpec((BN, C), lambda i: (i, 0)),
            pl.BlockSpec((BN, 2 * C), lambda i: (i, 0)),
            pl.BlockSpec((BN, C), lambda i: (i, 0)),
        ],
        out_shape=[
            jax.ShapeDtypeStruct((N, C), jnp.float32),
            jax.ShapeDtypeStruct((N, 2 * C), jnp.bfloat16),
            jax.ShapeDtypeStruct((N, C), jnp.float32),
        ],
    )(xh, W, b)


def _sum32_tc(s_all):
    def body(s_ref, o_ref):
        o_ref[...] = jnp.sum(s_ref[...], axis=0, keepdims=True) + 1e-16

    return pl.pallas_call(
        body,
        out_shape=jax.ShapeDtypeStruct((1, s_all.shape[1]), jnp.float32),
    )(s_all)


def _norm_project_tc(o_part, s_col, skip, W, b, inv_sqrt):
    _, N, C = o_part.shape
    C4 = W.shape[1]
    BN = 1000

    def body(o_ref, sc_ref, sk_ref, w_ref, b_ref, q_ref, kv_ref, s_ref):
        att = (o_ref[0] + o_ref[1]) / sc_ref[...]
        h = jax.nn.relu(att + sk_ref[...])
        acc = jnp.dot(h, w_ref[...],
                      preferred_element_type=jnp.float32) + b_ref[...]
        q_ref[...] = acc[:, 0:C] * inv_sqrt
        kv_ref[...] = acc[:, C:3 * C].astype(jnp.bfloat16)
        s_ref[...] = acc[:, 3 * C:4 * C]

    return pl.pallas_call(
        body,
        grid=(N // BN,),
        in_specs=[
            pl.BlockSpec((2, BN, C), lambda i: (0, i, 0)),
            pl.BlockSpec((BN, 1), lambda i: (i, 0)),
            pl.BlockSpec((BN, C), lambda i: (i, 0)),
            pl.BlockSpec((C, C4), lambda i: (0, 0)),
            pl.BlockSpec((1, C4), lambda i: (0, 0)),
        ],
        out_specs=[
            pl.BlockSpec((BN, C), lambda i: (i, 0)),
            pl.BlockSpec((BN, 2 * C), lambda i: (i, 0)),
            pl.BlockSpec((BN, C), lambda i: (i, 0)),
        ],
        out_shape=[
            jax.ShapeDtypeStruct((N, C), jnp.float32),
            jax.ShapeDtypeStruct((N, 2 * C), jnp.bfloat16),
            jax.ShapeDtypeStruct((N, C), jnp.float32),
        ],
    )(o_part, s_col, skip, W, b)


def _combine_tc(o_part, s_col, skip):
    _, N, C = o_part.shape
    BN = 1000

    def body(o_ref, sc_ref, sk_ref, out_ref):
        att = (o_ref[0] + o_ref[1]) / sc_ref[...]
        out_ref[...] = jax.nn.relu(att + sk_ref[...])

    return pl.pallas_call(
        body,
        grid=(N // BN,),
        in_specs=[
            pl.BlockSpec((2, BN, C), lambda i: (0, i, 0)),
            pl.BlockSpec((BN, 1), lambda i: (i, 0)),
            pl.BlockSpec((BN, C), lambda i: (i, 0)),
        ],
        out_specs=pl.BlockSpec((BN, C), lambda i: (i, 0)),
        out_shape=jax.ShapeDtypeStruct((N, C), jnp.float32),
    )(o_part, s_col, skip)


def _edge_layer_sc(qm, kvm, dstv, srcv, zeros_h):
    N, C = qm.shape
    C2 = C // 2
    E = dstv.shape[0]
    EPW = E // NW
    NCH = EPW // CH
    NG = CH // LANES
    NR = C // LANES
    SPAN = ((N + NSUB - 1) // NSUB + 7) // 8 * 8
    assert NCH % 2 == 1

    mesh = plsc.VectorSubcoreMesh(core_axis_name="c", subcore_axis_name="s")

    def body(q_hbm, kv_hbm, dst_hbm, src_hbm, z_hbm, out_part, s_all,
             dst_w, src_w, s_local, qb, kvb, vob, idxd,
             gsem0, gsem1, ssem0, ssem1, out_shared):
        cid = lax.axis_index("c")
        sid = lax.axis_index("s")
        wid = sid * NCORES + cid
        base = wid * EPW
        gsems = (gsem0, gsem1)
        ssems = (ssem0, ssem1)

        iota = lax.iota(jnp.int32, LANES)
        ziota = iota * 0
        zero = jnp.zeros((LANES,), jnp.float32)

        def fire_gathers(b, cc):
            cb = cc * CH
            pltpu.async_copy(
                q_hbm.at[dst_w.at[pl.ds(cb, CH)]], qb.at[b], gsems[b])
            pltpu.async_copy(
                kv_hbm.at[src_w.at[pl.ds(cb, CH)]], kvb.at[b], gsems[b])

        def wait_gathers(b):
            pltpu.make_async_copy(
                q_hbm.at[dst_w.at[pl.ds(0, CH)]], qb.at[b], gsems[b]).wait()
            pltpu.make_async_copy(
                kv_hbm.at[src_w.at[pl.ds(0, CH)]], kvb.at[b], gsems[b]).wait()

        def fire_scatter(b):
            pltpu.async_copy(
                vob.at[b], out_shared.at[idxd.at[b]], ssems[b], add=True)

        def wait_scatter(b):
            pltpu.make_async_copy(
                vob.at[b], out_shared.at[idxd.at[b]], ssems[b]).wait()

        def compute(b, cc):
            cb = cc * CH

            def edge4_dot(g, alphas):
                for u in range(4):
                    t = g * 4 + u
                    p = []
                    for r in range(C // 32):
                        qa = qb[b, t, pl.ds(r * 32, LANES)]
                        qc = qb[b, t, pl.ds(r * 32 + LANES, LANES)]
                        ka, kc = _unpack(kvb[b, t, pl.ds(r * LANES, LANES)])
                        p.append(qa * ka + qc * kc)
                    acc = (p[0] + p[1]) + (p[2] + p[3])
                    for sh in (8, 4, 2, 1):
                        acc = acc + _vgather(acc, iota ^ sh)
                    alphas = jnp.where(iota == t, acc, alphas)
                return alphas

            ev = jnp.exp(lax.fori_loop(0, LANES // 4, edge4_dot, zero))
            ks = dst_w[pl.ds(cb, LANES)]
            vs = ev
            mn = iota
            for r in range(1, LANES):
                perm = (iota + r) & (LANES - 1)
                same = _vgather(ks, perm) == ks
                vs = vs + jnp.where(same, _vgather(ev, perm), 0.0)
                mn = jnp.minimum(mn, jnp.where(same, perm, LANES))
            plsc.addupdate_scatter(s_local, [ks], vs, mask=mn == iota)
            idxd[b, pl.ds(0, LANES)] = ks
            def edge4_scale(g, scarry):
                for u in range(4):
                    t = g * 4 + u
                    av = _vgather(ev, ziota + t)
                    for r in range(C // 32):
                        va, vc = _unpack(
                            kvb[b, t, pl.ds(C2 + r * LANES, LANES)])
                        vob[b, t, pl.ds(r * 32, LANES)] = va * av
                        vob[b, t, pl.ds(r * 32 + LANES, LANES)] = vc * av
                return scarry

            lax.fori_loop(0, LANES // 4, edge4_scale, 0)

        pltpu.sync_copy(dst_hbm.at[pl.ds(base, EPW)], dst_w)
        pltpu.sync_copy(src_hbm.at[pl.ds(base, EPW)], src_w)

        def zloop(i, carry):
            s_local[pl.ds(i * LANES, LANES)] = zero
            return carry

        lax.fori_loop(0, N // LANES, zloop, 0)
        stripe = jnp.minimum(sid * SPAN, N - SPAN)
        pltpu.sync_copy(z_hbm.at[pl.ds(stripe, SPAN)],
                        out_shared.at[pl.ds(stripe, SPAN)])
        fire_gathers(0, 0)
        plsc.subcore_barrier()

        def pair(i, carry):
            for b in (0, 1):
                cc = 2 * i + b
                fire_gathers(b ^ 1, cc + 1)
                wait_gathers(b)

                @pl.when(cc >= 2)
                def _():
                    wait_scatter(b)

                compute(b, cc)
                fire_scatter(b)
            return carry

        lax.fori_loop(0, (NCH - 1) // 2, pair, 0)

        wait_gathers(0)
        wait_scatter(0)
        compute(0, NCH - 1)
        fire_scatter(0)
        wait_scatter(1)
        wait_scatter(0)
        plsc.subcore_barrier()
        pltpu.sync_copy(s_local, s_all.at[wid])
        pltpu.sync_copy(out_shared.at[pl.ds(stripe, SPAN)],
                        out_part.at[cid, pl.ds(stripe, SPAN)])

    return pl.kernel(
        body,
        out_type=[
            jax.ShapeDtypeStruct((NCORES, N, C), jnp.float32),
            jax.ShapeDtypeStruct((NW, N), jnp.float32),
        ],
        mesh=mesh,
        scratch_types=[
            pltpu.VMEM((EPW,), jnp.int32),
            pltpu.VMEM((EPW,), jnp.int32),
            pltpu.VMEM((N,), jnp.float32),
            pltpu.VMEM((2, CH, C), jnp.float32),
            pltpu.VMEM((2, CH, C), jnp.int32),
            pltpu.VMEM((2, CH, C), jnp.float32),
            pltpu.VMEM((2, CH), jnp.int32),
            pltpu.SemaphoreType.DMA,
            pltpu.SemaphoreType.DMA,
            pltpu.SemaphoreType.DMA,
            pltpu.SemaphoreType.DMA,
            pltpu.VMEM_SHARED((N, C), jnp.float32),
        ],
        compiler_params=pltpu.CompilerParams(needs_layout_passes=False),
    )(qm, kvm, dstv, srcv, zeros_h)


def kernel(x, edge_index, Wq0, bq0, Wk0, bk0, Wv0, bv0, Ws0, bs0,
           Wq1, bq1, Wk1, bk1, Wv1, bv1, Ws1, bs1):
    src = edge_index[0]
    dst = edge_index[1]
    N, _ = x.shape
    C = Wq0.shape[1]
    inv_sqrt = 1.0 / math.sqrt(C)

    i = jnp.arange(C)
    r32, w32 = i // 32, i % 32
    vperm = jnp.where(w32 % 2 == 0, r32 * 32 + w32 // 2,
                      r32 * 32 + 16 + (w32 - 1) // 2)
    qperm = jnp.where(w32 < 16, r32 * 32 + 2 * w32,
                      r32 * 32 + 2 * (w32 - 16) + 1)
    W0 = jnp.concatenate([Wq0[:, qperm], Wk0, Wv0[:, vperm], Ws0], axis=1)
    b0 = jnp.concatenate([bq0[qperm], bk0, bv0[vperm], bs0]).reshape(1, -1)
    W1 = jnp.concatenate([Wq1[:, qperm], Wk1, Wv1[:, vperm], Ws1], axis=1)
    b1 = jnp.concatenate([bq1[qperm], bk1, bv1[vperm], bs1]).reshape(1, -1)
    zeros_h = jnp.zeros((N, C), jnp.float32)

    q0, kv0, sk0 = _project_tc(x, W0, b0, inv_sqrt)
    op0, s_all0 = _edge_layer_sc(q0, _as_i32(kv0), dst, src, zeros_h)
    s_col0 = _sum32_tc(s_all0).reshape(N, 1)
    q1, kv1, sk1 = _norm_project_tc(op0, s_col0, sk0, W1, b1, inv_sqrt)
    op1, s_all1 = _edge_layer_sc(q1, _as_i32(kv1), dst, src, zeros_h)
    s_col1 = _sum32_tc(s_all1).reshape(N, 1)
    return _combine_tc(op1, s_col1, sk1)
